# baseline (speedup 1.0000x reference)
import functools

import jax
import jax.numpy as jnp
from jax import lax
from jax.experimental import pallas as pl
from jax.experimental.pallas import tpu as pltpu

_BF = jnp.bfloat16
_F32 = jnp.float32
_F8 = jnp.float8_e4m3fn
_GELU_C1 = 0.7978845608028654
_GELU_C3 = 0.7978845608028654 * 0.044715


def _dot(a, b):
    return jax.lax.dot_general(
        a, b, (((1,), (0,)), ((), ())), preferred_element_type=_F32
    )


def _group_rms_scale(x, g1_ref, g1t_ref, scale_ref, inv_dh):
    sq = (x * x).astype(_BF)
    gs = _dot(sq, g1_ref[...])
    bsum = _dot(gs.astype(_BF), g1t_ref[...])
    return x * lax.rsqrt(bsum * inv_dh + 1e-6) * scale_ref[...]


def _ln_qkv_body(D, inv_dh, x_ref, g_ref, b_ref, wall_ref, gq_ref, gqt_ref,
                 qgt_ref, kgt_ref, q_ref, k_ref, v_ref):
    x = x_ref[...]
    mu = jnp.mean(x, axis=-1, keepdims=True)
    xc = x - mu
    var = jnp.mean(xc * xc, axis=-1, keepdims=True)
    hn = xc * lax.rsqrt(var + 1e-5) * g_ref[...] + b_ref[...]
    qkv = _dot(hn.astype(_BF), wall_ref[...])
    q = qkv[:, :D]
    k = qkv[:, D:2 * D]
    v = qkv[:, 2 * D:]
    q_ref[...] = _group_rms_scale(q, gq_ref, gqt_ref, qgt_ref,
                                  inv_dh).astype(_BF)
    k_ref[...] = _group_rms_scale(k, gq_ref, gqt_ref, kgt_ref,
                                  inv_dh).astype(_BF)
    v_ref[...] = v.astype(_BF)


def _attn_body(qoff, BQ, KS, q_ref, kt_ref, v_ref, o_ref):
    i = pl.program_id(1)
    row = lax.broadcasted_iota(jnp.int32, (BQ, KS), 0) + (qoff + i * BQ)
    col = lax.broadcasted_iota(jnp.int32, (BQ, KS), 1)
    keep = col <= row
    for hh in range(2):
        s = _dot(q_ref[hh], kt_ref[hh])
        p = jnp.exp(jnp.where(keep, s, -30.0))
        denom = jnp.sum(p, axis=-1, keepdims=True)
        o = _dot(p.astype(_BF), v_ref[hh])
        o_ref[hh] = (o / denom).astype(_BF)


def _proj_gate_body(x_ref, o_ref, wo_ref, wg_ref, x1_ref, x1f_ref, gate_ref):
    x1 = x_ref[...] + _dot(o_ref[...], wo_ref[...])
    x1_ref[...] = x1
    x1f_ref[...] = x1.astype(_F8)
    logits = _dot(x1.astype(_BF), wg_ref[...])
    m = jnp.max(logits, axis=-1, keepdims=True)
    p = jnp.exp(logits - m)
    gate_ref[...] = p / jnp.sum(p, axis=-1, keepdims=True)


def _moe_body(E, x1f_ref, gate_ref, x1_ref, w1_ref, w2_ref, y_ref):
    ep = pl.program_id(0)
    f = pl.program_id(1)

    @pl.when((ep == 0) & (f == 0))
    def _init():
        y_ref[...] = x1_ref[...]

    gate = gate_ref[...]
    col = lax.broadcasted_iota(jnp.int32, gate.shape, 1)
    x1f = x1f_ref[...]
    acc = None
    for ee in range(2):
        Hb = _dot(x1f, w1_ref[ee]).astype(_BF)
        g = jnp.sum(jnp.where(col == ep * 2 + ee, gate, 0.0),
                    axis=-1, keepdims=True)
        gb = (g * 0.25).astype(_BF)
        t = Hb * (_GELU_C3 / 262144.0 * (Hb * Hb) + _GELU_C1 / 64.0)
        th = jnp.tanh(t)
        sgh = gb * Hb
        hg = (sgh + sgh * th).astype(_F8)
        d = _dot(hg, w2_ref[ee])
        acc = d if acc is None else acc + d
    y_ref[...] += acc * (1.0 / (32.0 * 64.0))


def kernel(x, ln_g, ln_b, Wq, Wk, Wv, Wo, qg, kg, Wg, w1, w2):
    B, S, D = x.shape
    DH = qg.shape[0]
    H = Wq.shape[1] // DH
    E = Wg.shape[1]
    FF = w1.shape[2]

    xs = x.reshape(S, D)
    W_all = jnp.concatenate([Wq, Wk, Wv], axis=1).astype(_BF)
    Gq = (jnp.arange(D)[:, None] // DH == jnp.arange(H)[None, :]).astype(_BF)
    GqT = Gq.T
    qg_t = (jnp.tile(qg, H) * (DH ** -0.5)).reshape(1, D)
    kg_t = jnp.tile(kg, H).reshape(1, D)
    Wob = Wo.astype(_BF)
    Wgb = Wg.astype(_BF)
    w1f = (w1 * 64.0).astype(_F8)
    w2f = (w2 * 64.0).astype(_F8)
    g2 = ln_g.reshape(1, D)
    b2 = ln_b.reshape(1, D)

    BS1 = 512
    q2, k2, v2 = pl.pallas_call(
        functools.partial(_ln_qkv_body, D, 1.0 / DH),
        grid=(S // BS1,),
        in_specs=[
            pl.BlockSpec((BS1, D), lambda i: (i, 0)),
            pl.BlockSpec((1, D), lambda i: (0, 0)),
            pl.BlockSpec((1, D), lambda i: (0, 0)),
            pl.BlockSpec((D, 3 * D), lambda i: (0, 0)),
            pl.BlockSpec((D, H), lambda i: (0, 0)),
            pl.BlockSpec((H, D), lambda i: (0, 0)),
            pl.BlockSpec((1, D), lambda i: (0, 0)),
            pl.BlockSpec((1, D), lambda i: (0, 0)),
        ],
        out_specs=[
            pl.BlockSpec((BS1, D), lambda i: (i, 0)),
            pl.BlockSpec((BS1, D), lambda i: (i, 0)),
            pl.BlockSpec((BS1, D), lambda i: (i, 0)),
        ],
        out_shape=[jax.ShapeDtypeStruct((S, D), _BF)] * 3,
        compiler_params=pltpu.CompilerParams(
            dimension_semantics=("parallel",)
        ),
    )(xs, g2, b2, W_all, Gq, GqT, qg_t, kg_t)

    q3 = q2.reshape(S, H, DH).transpose(1, 0, 2)
    k3t = k2.reshape(S, H, DH).transpose(1, 2, 0)
    v3 = v2.reshape(S, H, DH).transpose(1, 0, 2)

    HALF = S // 2
    BQ = 512
    NQ = HALF // BQ

    def attn_call(qoff, KS):
        qb = qoff // BQ
        return pl.pallas_call(
            functools.partial(_attn_body, qoff, BQ, KS),
            grid=(H // 2, NQ),
            in_specs=[
                pl.BlockSpec((2, BQ, DH), lambda hp, i: (hp, qb + i, 0)),
                pl.BlockSpec((2, DH, KS), lambda hp, i: (hp, 0, 0)),
                pl.BlockSpec((2, KS, DH), lambda hp, i: (hp, 0, 0)),
            ],
            out_specs=pl.BlockSpec((2, BQ, DH), lambda hp, i: (hp, i, 0)),
            out_shape=jax.ShapeDtypeStruct((H, HALF, DH), _BF),
            compiler_params=pltpu.CompilerParams(
                dimension_semantics=("parallel", "parallel")
            ),
        )(q3, k3t, v3)

    o_lo = attn_call(0, HALF)
    o_hi = attn_call(HALF, S)
    o2 = (
        jnp.concatenate([o_lo, o_hi], axis=1)
        .transpose(1, 0, 2)
        .reshape(S, H * DH)
    )

    BS3 = 512
    x1, x1f, gate = pl.pallas_call(
        _proj_gate_body,
        grid=(S // BS3,),
        in_specs=[
            pl.BlockSpec((BS3, D), lambda i: (i, 0)),
            pl.BlockSpec((BS3, H * DH), lambda i: (i, 0)),
            pl.BlockSpec((H * DH, D), lambda i: (0, 0)),
            pl.BlockSpec((D, E), lambda i: (0, 0)),
        ],
        out_specs=[
            pl.BlockSpec((BS3, D), lambda i: (i, 0)),
            pl.BlockSpec((BS3, D), lambda i: (i, 0)),
            pl.BlockSpec((BS3, E), lambda i: (i, 0)),
        ],
        out_shape=[
            jax.ShapeDtypeStruct((S, D), _F32),
            jax.ShapeDtypeStruct((S, D), _F8),
            jax.ShapeDtypeStruct((S, E), _F32),
        ],
        compiler_params=pltpu.CompilerParams(
            dimension_semantics=("parallel",)
        ),
    )(xs, o2, Wob, Wgb)

    FFB = 1536
    NF = FF // FFB
    y = pl.pallas_call(
        functools.partial(_moe_body, E),
        grid=(E // 2, NF),
        in_specs=[
            pl.BlockSpec((S, D), lambda e, f: (0, 0)),
            pl.BlockSpec((S, E), lambda e, f: (0, 0)),
            pl.BlockSpec((S, D), lambda e, f: (0, 0)),
            pl.BlockSpec((2, D, FFB), lambda e, f: (e, 0, f)),
            pl.BlockSpec((2, FFB, D), lambda e, f: (e, f, 0)),
        ],
        out_specs=pl.BlockSpec((S, D), lambda e, f: (0, 0)),
        out_shape=jax.ShapeDtypeStruct((S, D), _F32),
        compiler_params=pltpu.CompilerParams(
            dimension_semantics=("arbitrary", "arbitrary")
        ),
    )(x1f, gate, x1, w1f, w2f)

    return y.reshape(B, S, D)


# speedup vs baseline: 1.0274x; 1.0274x over previous
import functools

import jax
import jax.numpy as jnp
from jax import lax
from jax.experimental import pallas as pl
from jax.experimental.pallas import tpu as pltpu

_BF = jnp.bfloat16
_F32 = jnp.float32
_F8 = jnp.float8_e4m3fn
_GELU_C1 = 0.7978845608028654
_GELU_C3 = 0.7978845608028654 * 0.044715


def _dot(a, b):
    return jax.lax.dot_general(
        a, b, (((1,), (0,)), ((), ())), preferred_element_type=_F32
    )


def _group_rms_scale(x, g1_ref, g1t_ref, scale_ref, inv_dh):
    sq = (x * x).astype(_BF)
    gs = _dot(sq, g1_ref[...])
    bsum = _dot(gs.astype(_BF), g1t_ref[...])
    return x * lax.rsqrt(bsum * inv_dh + 1e-6) * scale_ref[...]


def _ln_qkv_body(D, inv_dh, x_ref, g_ref, b_ref, wall_ref, gq_ref, gqt_ref,
                 qgt_ref, kgt_ref, q_ref, k_ref, v_ref):
    x = x_ref[...]
    mu = jnp.mean(x, axis=-1, keepdims=True)
    xc = x - mu
    var = jnp.mean(xc * xc, axis=-1, keepdims=True)
    hn = xc * lax.rsqrt(var + 1e-5) * g_ref[...] + b_ref[...]
    qkv = _dot(hn.astype(_BF), wall_ref[...])
    q = qkv[:, :D]
    k = qkv[:, D:2 * D]
    v = qkv[:, 2 * D:]
    q_ref[...] = _group_rms_scale(q, gq_ref, gqt_ref, qgt_ref,
                                  inv_dh).astype(_BF)
    k_ref[...] = _group_rms_scale(k, gq_ref, gqt_ref, kgt_ref,
                                  inv_dh).astype(_BF)
    v_ref[...] = v.astype(_BF)


def _attn_body(qoff, BQ, KS, q_ref, kt_ref, v_ref, o_ref):
    i = pl.program_id(1)
    row = lax.broadcasted_iota(jnp.int32, (BQ, KS), 0) + (qoff + i * BQ)
    col = lax.broadcasted_iota(jnp.int32, (BQ, KS), 1)
    keep = col <= row
    for hh in range(2):
        s = _dot(q_ref[hh], kt_ref[hh])
        p = jnp.exp(jnp.where(keep, s, -30.0))
        denom = jnp.sum(p, axis=-1, keepdims=True)
        o = _dot(p.astype(_BF), v_ref[hh])
        o_ref[hh] = (o / denom).astype(_BF)


def _proj_gate_body(x_ref, o_ref, wo_ref, wg_ref, x1_ref, x1f_ref, gate_ref):
    x1 = x_ref[...] + _dot(o_ref[...], wo_ref[...])
    x1_ref[...] = x1
    x1f_ref[...] = x1.astype(_F8)
    logits = _dot(x1.astype(_BF), wg_ref[...])
    m = jnp.max(logits, axis=-1, keepdims=True)
    p = jnp.exp(logits - m)
    gate_ref[...] = p / jnp.sum(p, axis=-1, keepdims=True)


def _moe_body(E, x1f_ref, gate_ref, x1_ref, w1_ref, w2_ref, y_ref):
    e = pl.program_id(0)
    f = pl.program_id(1)

    @pl.when((e == 0) & (f == 0))
    def _init():
        y_ref[...] = x1_ref[...]

    Hb = _dot(x1f_ref[...], w1_ref[0]).astype(_BF)
    gate = gate_ref[...]
    col = lax.broadcasted_iota(jnp.int32, gate.shape, 1)
    g = jnp.sum(jnp.where(col == e, gate, 0.0), axis=-1, keepdims=True)
    gb = (g * 0.25).astype(_BF)
    t = Hb * (_GELU_C3 / 262144.0 * (Hb * Hb) + _GELU_C1 / 64.0)
    th = jnp.tanh(t)
    sgh = gb * Hb
    hg = (sgh + sgh * th).astype(_F8)
    y_ref[...] += _dot(hg, w2_ref[0]) * (1.0 / (32.0 * 64.0))


def kernel(x, ln_g, ln_b, Wq, Wk, Wv, Wo, qg, kg, Wg, w1, w2):
    B, S, D = x.shape
    DH = qg.shape[0]
    H = Wq.shape[1] // DH
    E = Wg.shape[1]
    FF = w1.shape[2]

    xs = x.reshape(S, D)
    W_all = jnp.concatenate([Wq, Wk, Wv], axis=1).astype(_BF)
    Gq = (jnp.arange(D)[:, None] // DH == jnp.arange(H)[None, :]).astype(_BF)
    GqT = Gq.T
    qg_t = (jnp.tile(qg, H) * (DH ** -0.5)).reshape(1, D)
    kg_t = jnp.tile(kg, H).reshape(1, D)
    Wob = Wo.astype(_BF)
    Wgb = Wg.astype(_BF)
    w1f = (w1 * 64.0).astype(_F8)
    w2f = (w2 * 64.0).astype(_F8)
    g2 = ln_g.reshape(1, D)
    b2 = ln_b.reshape(1, D)

    BS1 = 512
    q2, k2, v2 = pl.pallas_call(
        functools.partial(_ln_qkv_body, D, 1.0 / DH),
        grid=(S // BS1,),
        in_specs=[
            pl.BlockSpec((BS1, D), lambda i: (i, 0)),
            pl.BlockSpec((1, D), lambda i: (0, 0)),
            pl.BlockSpec((1, D), lambda i: (0, 0)),
            pl.BlockSpec((D, 3 * D), lambda i: (0, 0)),
            pl.BlockSpec((D, H), lambda i: (0, 0)),
            pl.BlockSpec((H, D), lambda i: (0, 0)),
            pl.BlockSpec((1, D), lambda i: (0, 0)),
            pl.BlockSpec((1, D), lambda i: (0, 0)),
        ],
        out_specs=[
            pl.BlockSpec((BS1, D), lambda i: (i, 0)),
            pl.BlockSpec((BS1, D), lambda i: (i, 0)),
            pl.BlockSpec((BS1, D), lambda i: (i, 0)),
        ],
        out_shape=[jax.ShapeDtypeStruct((S, D), _BF)] * 3,
        compiler_params=pltpu.CompilerParams(
            dimension_semantics=("parallel",)
        ),
    )(xs, g2, b2, W_all, Gq, GqT, qg_t, kg_t)

    q3 = q2.reshape(S, H, DH).transpose(1, 0, 2)
    k3t = k2.reshape(S, H, DH).transpose(1, 2, 0)
    v3 = v2.reshape(S, H, DH).transpose(1, 0, 2)

    HALF = S // 2
    BQ = 1024
    NQ = HALF // BQ

    def attn_call(qoff, KS):
        qb = qoff // BQ
        return pl.pallas_call(
            functools.partial(_attn_body, qoff, BQ, KS),
            grid=(H // 2, NQ),
            in_specs=[
                pl.BlockSpec((2, BQ, DH), lambda hp, i: (hp, qb + i, 0)),
                pl.BlockSpec((2, DH, KS), lambda hp, i: (hp, 0, 0)),
                pl.BlockSpec((2, KS, DH), lambda hp, i: (hp, 0, 0)),
            ],
            out_specs=pl.BlockSpec((2, BQ, DH), lambda hp, i: (hp, i, 0)),
            out_shape=jax.ShapeDtypeStruct((H, HALF, DH), _BF),
            compiler_params=pltpu.CompilerParams(
                dimension_semantics=("parallel", "parallel")
            ),
        )(q3, k3t, v3)

    o_lo = attn_call(0, HALF)
    o_hi = attn_call(HALF, S)
    o2 = (
        jnp.concatenate([o_lo, o_hi], axis=1)
        .transpose(1, 0, 2)
        .reshape(S, H * DH)
    )

    BS3 = 512
    x1, x1f, gate = pl.pallas_call(
        _proj_gate_body,
        grid=(S // BS3,),
        in_specs=[
            pl.BlockSpec((BS3, D), lambda i: (i, 0)),
            pl.BlockSpec((BS3, H * DH), lambda i: (i, 0)),
            pl.BlockSpec((H * DH, D), lambda i: (0, 0)),
            pl.BlockSpec((D, E), lambda i: (0, 0)),
        ],
        out_specs=[
            pl.BlockSpec((BS3, D), lambda i: (i, 0)),
            pl.BlockSpec((BS3, D), lambda i: (i, 0)),
            pl.BlockSpec((BS3, E), lambda i: (i, 0)),
        ],
        out_shape=[
            jax.ShapeDtypeStruct((S, D), _F32),
            jax.ShapeDtypeStruct((S, D), _F8),
            jax.ShapeDtypeStruct((S, E), _F32),
        ],
        compiler_params=pltpu.CompilerParams(
            dimension_semantics=("parallel",)
        ),
    )(xs, o2, Wob, Wgb)

    FFB = 1536
    NF = FF // FFB
    y = pl.pallas_call(
        functools.partial(_moe_body, E),
        grid=(E, NF),
        in_specs=[
            pl.BlockSpec((S, D), lambda e, f: (0, 0)),
            pl.BlockSpec((S, E), lambda e, f: (0, 0)),
            pl.BlockSpec((S, D), lambda e, f: (0, 0)),
            pl.BlockSpec((1, D, FFB), lambda e, f: (e, 0, f)),
            pl.BlockSpec((1, FFB, D), lambda e, f: (e, f, 0)),
        ],
        out_specs=pl.BlockSpec((S, D), lambda e, f: (0, 0)),
        out_shape=jax.ShapeDtypeStruct((S, D), _F32),
        compiler_params=pltpu.CompilerParams(
            dimension_semantics=("arbitrary", "arbitrary")
        ),
    )(x1f, gate, x1, w1f, w2f)

    return y.reshape(B, S, D)


# speedup vs baseline: 1.0303x; 1.0028x over previous
import functools

import jax
import jax.numpy as jnp
from jax import lax
from jax.experimental import pallas as pl
from jax.experimental.pallas import tpu as pltpu

_BF = jnp.bfloat16
_F32 = jnp.float32
_F8 = jnp.float8_e4m3fn
_GELU_C1 = 0.7978845608028654
_GELU_C3 = 0.7978845608028654 * 0.044715


def _dot(a, b):
    return jax.lax.dot_general(
        a, b, (((1,), (0,)), ((), ())), preferred_element_type=_F32
    )


def _group_rms_scale(x, g1_ref, g1t_ref, scale_ref, inv_dh):
    sq = (x * x).astype(_BF)
    gs = _dot(sq, g1_ref[...])
    bsum = _dot(gs.astype(_BF), g1t_ref[...])
    return x * lax.rsqrt(bsum * inv_dh + 1e-6) * scale_ref[...]


def _ln_qkv_body(D, inv_dh, x_ref, g_ref, b_ref, wall_ref, gq_ref, gqt_ref,
                 qgt_ref, kgt_ref, q_ref, k_ref, v_ref):
    x = x_ref[...]
    mu = jnp.mean(x, axis=-1, keepdims=True)
    xc = x - mu
    var = jnp.mean(xc * xc, axis=-1, keepdims=True)
    hn = xc * lax.rsqrt(var + 1e-5) * g_ref[...] + b_ref[...]
    qkv = _dot(hn.astype(_BF), wall_ref[...])
    q = qkv[:, :D]
    k = qkv[:, D:2 * D]
    v = qkv[:, 2 * D:]
    q_ref[...] = _group_rms_scale(q, gq_ref, gqt_ref, qgt_ref,
                                  inv_dh).astype(_BF)
    k_ref[...] = _group_rms_scale(k, gq_ref, gqt_ref, kgt_ref,
                                  inv_dh).astype(_BF)
    v_ref[...] = v.astype(_BF)


def _attn_body(qoff, BQ, KS, q_ref, kt_ref, v_ref, o_ref):
    i = pl.program_id(1)
    row = lax.broadcasted_iota(jnp.int32, (BQ, KS), 0) + (qoff + i * BQ)
    col = lax.broadcasted_iota(jnp.int32, (BQ, KS), 1)
    keep = col <= row
    for hh in range(2):
        s = _dot(q_ref[hh], kt_ref[hh])
        p = jnp.exp(jnp.where(keep, s, -30.0))
        denom = jnp.sum(p, axis=-1, keepdims=True)
        o = _dot(p.astype(_BF), v_ref[hh])
        o_ref[hh] = (o / denom).astype(_BF)


def _proj_gate_body(x_ref, o_ref, wo_ref, wg_ref, x1_ref, x1f_ref, gate_ref):
    x1 = x_ref[...] + _dot(o_ref[...], wo_ref[...])
    x1_ref[...] = x1
    x1f_ref[...] = x1.astype(_F8)
    logits = _dot(x1.astype(_BF), wg_ref[...])
    m = jnp.max(logits, axis=-1, keepdims=True)
    p = jnp.exp(logits - m)
    gate_ref[...] = p / jnp.sum(p, axis=-1, keepdims=True)


def _moe_body(E, x1f_ref, gate_ref, x1_ref, w1_ref, w2_ref, y_ref):
    e = pl.program_id(0)
    f = pl.program_id(1)

    @pl.when((e == 0) & (f == 0))
    def _init():
        y_ref[...] = x1_ref[...]

    Hb = _dot(x1f_ref[...], w1_ref[0]).astype(_BF)
    gate = gate_ref[...]
    col = lax.broadcasted_iota(jnp.int32, gate.shape, 1)
    g = jnp.sum(jnp.where(col == e, gate, 0.0), axis=-1, keepdims=True)
    gb = (g * 0.25).astype(_BF)
    t = Hb * (_GELU_C3 / 262144.0 * (Hb * Hb) + _GELU_C1 / 64.0)
    th = jnp.tanh(t)
    sgh = gb * Hb
    hg = (sgh + sgh * th).astype(_F8)
    y_ref[...] += _dot(hg, w2_ref[0]) * (1.0 / (32.0 * 64.0))


def kernel(x, ln_g, ln_b, Wq, Wk, Wv, Wo, qg, kg, Wg, w1, w2):
    B, S, D = x.shape
    DH = qg.shape[0]
    H = Wq.shape[1] // DH
    E = Wg.shape[1]
    FF = w1.shape[2]

    xs = x.reshape(S, D)
    W_all = jnp.concatenate([Wq, Wk, Wv], axis=1).astype(_BF)
    Gq = (jnp.arange(D)[:, None] // DH == jnp.arange(H)[None, :]).astype(_BF)
    GqT = Gq.T
    qg_t = (jnp.tile(qg, H) * (DH ** -0.5)).reshape(1, D)
    kg_t = jnp.tile(kg, H).reshape(1, D)
    Wob = Wo.astype(_BF)
    Wgb = Wg.astype(_BF)
    w1f = (w1 * 64.0).astype(_F8)
    w2f = (w2 * 64.0).astype(_F8)
    g2 = ln_g.reshape(1, D)
    b2 = ln_b.reshape(1, D)

    BS1 = 1024
    q2, k2, v2 = pl.pallas_call(
        functools.partial(_ln_qkv_body, D, 1.0 / DH),
        grid=(S // BS1,),
        in_specs=[
            pl.BlockSpec((BS1, D), lambda i: (i, 0)),
            pl.BlockSpec((1, D), lambda i: (0, 0)),
            pl.BlockSpec((1, D), lambda i: (0, 0)),
            pl.BlockSpec((D, 3 * D), lambda i: (0, 0)),
            pl.BlockSpec((D, H), lambda i: (0, 0)),
            pl.BlockSpec((H, D), lambda i: (0, 0)),
            pl.BlockSpec((1, D), lambda i: (0, 0)),
            pl.BlockSpec((1, D), lambda i: (0, 0)),
        ],
        out_specs=[
            pl.BlockSpec((BS1, D), lambda i: (i, 0)),
            pl.BlockSpec((BS1, D), lambda i: (i, 0)),
            pl.BlockSpec((BS1, D), lambda i: (i, 0)),
        ],
        out_shape=[jax.ShapeDtypeStruct((S, D), _BF)] * 3,
        compiler_params=pltpu.CompilerParams(
            dimension_semantics=("parallel",)
        ),
    )(xs, g2, b2, W_all, Gq, GqT, qg_t, kg_t)

    q3 = q2.reshape(S, H, DH).transpose(1, 0, 2)
    k3t = k2.reshape(S, H, DH).transpose(1, 2, 0)
    v3 = v2.reshape(S, H, DH).transpose(1, 0, 2)

    HALF = S // 2
    BQ = 1024
    NQ = HALF // BQ

    def attn_call(qoff, KS):
        qb = qoff // BQ
        return pl.pallas_call(
            functools.partial(_attn_body, qoff, BQ, KS),
            grid=(H // 2, NQ),
            in_specs=[
                pl.BlockSpec((2, BQ, DH), lambda hp, i: (hp, qb + i, 0)),
                pl.BlockSpec((2, DH, KS), lambda hp, i: (hp, 0, 0)),
                pl.BlockSpec((2, KS, DH), lambda hp, i: (hp, 0, 0)),
            ],
            out_specs=pl.BlockSpec((2, BQ, DH), lambda hp, i: (hp, i, 0)),
            out_shape=jax.ShapeDtypeStruct((H, HALF, DH), _BF),
            compiler_params=pltpu.CompilerParams(
                dimension_semantics=("parallel", "parallel")
            ),
        )(q3, k3t, v3)

    o_lo = attn_call(0, HALF)
    o_hi = attn_call(HALF, S)
    o2 = (
        jnp.concatenate([o_lo, o_hi], axis=1)
        .transpose(1, 0, 2)
        .reshape(S, H * DH)
    )

    BS3 = 1024
    x1, x1f, gate = pl.pallas_call(
        _proj_gate_body,
        grid=(S // BS3,),
        in_specs=[
            pl.BlockSpec((BS3, D), lambda i: (i, 0)),
            pl.BlockSpec((BS3, H * DH), lambda i: (i, 0)),
            pl.BlockSpec((H * DH, D), lambda i: (0, 0)),
            pl.BlockSpec((D, E), lambda i: (0, 0)),
        ],
        out_specs=[
            pl.BlockSpec((BS3, D), lambda i: (i, 0)),
            pl.BlockSpec((BS3, D), lambda i: (i, 0)),
            pl.BlockSpec((BS3, E), lambda i: (i, 0)),
        ],
        out_shape=[
            jax.ShapeDtypeStruct((S, D), _F32),
            jax.ShapeDtypeStruct((S, D), _F8),
            jax.ShapeDtypeStruct((S, E), _F32),
        ],
        compiler_params=pltpu.CompilerParams(
            dimension_semantics=("parallel",)
        ),
    )(xs, o2, Wob, Wgb)

    FFB = 1536
    NF = FF // FFB
    y = pl.pallas_call(
        functools.partial(_moe_body, E),
        grid=(E, NF),
        in_specs=[
            pl.BlockSpec((S, D), lambda e, f: (0, 0)),
            pl.BlockSpec((S, E), lambda e, f: (0, 0)),
            pl.BlockSpec((S, D), lambda e, f: (0, 0)),
            pl.BlockSpec((1, D, FFB), lambda e, f: (e, 0, f)),
            pl.BlockSpec((1, FFB, D), lambda e, f: (e, f, 0)),
        ],
        out_specs=pl.BlockSpec((S, D), lambda e, f: (0, 0)),
        out_shape=jax.ShapeDtypeStruct((S, D), _F32),
        compiler_params=pltpu.CompilerParams(
            dimension_semantics=("arbitrary", "arbitrary")
        ),
    )(x1f, gate, x1, w1f, w2f)

    return y.reshape(B, S, D)
